# baseline (speedup 1.0000x reference)
import functools

import jax
import jax.numpy as jnp
from jax import lax
from jax.experimental import pallas as pl
from jax.experimental.pallas import tpu as pltpu
from jax.experimental.pallas import tpu_sc as plsc

B, N, E = 4, 8192, 768
NKEEP = 4915
KPAD = 5120
NH = 7
NHEADS = 4

RI = 256
RJ = 2048
PB = 512

_NC, _NS = 2, 16
NW = _NC * _NS
ROWS_W = (B * KPAD) // NW
CH = 80
NCHUNK = ROWS_W // CH


def _rank_kernel(s_ref, st_ref, out_ref):
    j = pl.program_id(2)
    i = pl.program_id(1)
    si = st_ref[0]
    sj = s_ref[0]
    gt = sj > si
    eq = sj == si
    iidx = lax.broadcasted_iota(jnp.int32, (RI, RJ), 0) + i * RI
    jidx = lax.broadcasted_iota(jnp.int32, (RI, RJ), 1) + j * RJ
    before = gt | (eq & (jidx < iidx))
    cnt = jnp.sum(jnp.where(before, 1.0, 0.0), axis=1)

    @pl.when(j == 0)
    def _():
        out_ref[...] = jnp.zeros_like(out_ref)

    out_ref[...] += cnt[None, None, :]


def _invert_kernel(rank_ref, out_ref):
    bi = pl.program_id(0)
    p = pl.program_id(1)
    ranks = rank_ref[0].astype(jnp.int32)
    pidx = lax.broadcasted_iota(jnp.int32, (PB, N), 0) + p * PB
    toks = lax.broadcasted_iota(jnp.int32, (PB, N), 1)
    val = jnp.sum(jnp.where(ranks == pidx, toks, 0), axis=1)
    out_ref[...] = (val + bi * N)[None, None, :]


def _sc_gather_body(xflat, gidx, ctab, xout, cout,
                    idx_v, rows_v, ci0_v, ci1_v, o0_v, o1_v, sem):
    wid = lax.axis_index("s") * _NC + lax.axis_index("c")
    bi = wid // (NW // B)
    out_base = wid * ROWS_W
    q_base = (wid % (NW // B)) * ROWS_W

    for c in range(NCHUNK):
        pltpu.sync_copy(gidx.at[pl.ds(out_base + c * CH, CH)], idx_v.at[c])

    for c in range(NCHUNK):
        for k in range(CH // 16):
            iv = idx_v[c, pl.ds(k * 16, 16)]
            ci0_v[c, pl.ds(k * 16, 16)] = iv + bi * N
            ci1_v[c, pl.ds(k * 16, 16)] = iv + (bi + 1) * N

    for c in range(NCHUNK):
        pltpu.async_copy(xflat.at[idx_v.at[c]], rows_v, sem).wait()
        pltpu.sync_copy(rows_v, xout.at[pl.ds(out_base + c * CH, CH)])
        pltpu.async_copy(ctab.at[ci0_v.at[c]], o0_v.at[c], sem).wait()
        pltpu.async_copy(ctab.at[ci1_v.at[c]], o1_v.at[c], sem).wait()
        pltpu.sync_copy(
            o0_v.at[c],
            cout.at[pl.ds((bi * 2) * KPAD + q_base + c * CH, CH)])
        pltpu.sync_copy(
            o1_v.at[c],
            cout.at[pl.ds((bi * 2 + 1) * KPAD + q_base + c * CH, CH)])


@functools.partial(
    pl.kernel,
    mesh=plsc.VectorSubcoreMesh(core_axis_name="c", subcore_axis_name="s"),
    out_type=[
        jax.ShapeDtypeStruct((B * KPAD, E), jnp.float32),
        jax.ShapeDtypeStruct((B * 2 * KPAD,), jnp.float32),
    ],
    scratch_types=[
        pltpu.VMEM((NCHUNK, CH), jnp.int32),
        pltpu.VMEM((CH, E), jnp.float32),
        pltpu.VMEM((NCHUNK, CH), jnp.int32),
        pltpu.VMEM((NCHUNK, CH), jnp.int32),
        pltpu.VMEM((NCHUNK, CH), jnp.float32),
        pltpu.VMEM((NCHUNK, CH), jnp.float32),
        pltpu.SemaphoreType.DMA,
    ],
)
def _sc_gather(xflat, gidx, ctab, xout, cout, *scratch):
    _sc_gather_body(xflat, gidx, ctab, xout, cout, *scratch)


def kernel(x, att_nh, coords, W1, b1, W2, b2, W3, b3, W4, b4, W5, b5):
    b, n, e = x.shape
    bt, n_heads, nh, _ = att_nh.shape

    a = att_nh.reshape(b, n, nh, nh, n_heads)
    h = a @ W1.T + b1
    h = h @ W2.T + b2
    att_vote = h.reshape(b, n, nh * nh)
    v = att_vote @ W3.T + b3
    v = v @ W4.T + b4
    v = v @ W5.T + b5
    v = jnp.squeeze(v)
    s = jax.nn.softmax(v, axis=1)

    st = s[:, :, None]
    s3 = s[:, None, :]

    rank = pl.pallas_call(
        _rank_kernel,
        grid=(B, N // RI, N // RJ),
        in_specs=[
            pl.BlockSpec((1, 1, RJ), lambda bi, i, j: (bi, 0, j)),
            pl.BlockSpec((1, RI, 1), lambda bi, i, j: (bi, i, 0)),
        ],
        out_specs=pl.BlockSpec((1, 1, RI), lambda bi, i, j: (bi, 0, i)),
        out_shape=jax.ShapeDtypeStruct((B, 1, N), jnp.float32),
    )(s3, st)

    gidx = pl.pallas_call(
        _invert_kernel,
        grid=(B, KPAD // PB),
        in_specs=[pl.BlockSpec((1, 1, N), lambda bi, p: (bi, 0, 0))],
        out_specs=pl.BlockSpec((1, 1, PB), lambda bi, p: (bi, 0, p)),
        out_shape=jax.ShapeDtypeStruct((B, 1, KPAD), jnp.int32),
    )(rank)

    xflat = x.reshape(b * n, e)
    ctab = coords[..., 0].reshape(B * 2 * N)
    xout, cout = _sc_gather(xflat, gidx.reshape(B * KPAD), ctab)

    x_out = xout.reshape(B, KPAD, E)[:, :NKEEP]
    coords_out = cout.reshape(B, 2, KPAD)[:, :, :NKEEP, None]
    return (x_out, coords_out)
